# baseline (speedup 1.0000x reference)
import jax
import jax.numpy as jnp
from jax import lax
from jax.experimental import pallas as pl
from jax.experimental.pallas import tpu as pltpu
from jax.experimental.pallas import tpu_sc as plsc

N_NODES = 10000
D = 128
NQ = 4
Q = D // NQ
N_GRAPHS = 64

NC = 2
NS = 16
P = 10240
EPT = 20480
NCHUNK = 80
CW = 256
ROWS_PT = P // NS

_f32 = jnp.float32


NBUF = 4


def _sc_agg_body(xs_hbm, src_hbm, dst_hbm, out_hbm,
                 idx_s, idx_d, rows, zbuf, gsem, ssem, x_sh, agg_sh):
    c = lax.axis_index("c")
    s = lax.axis_index("s")

    pltpu.sync_copy(src_hbm.at[c, s], idx_s)
    pltpu.sync_copy(dst_hbm.at[c, s], idx_d)

    def _zf(i, carry):
        zbuf[i // 2, pl.ds((i % 2) * 16, 16)] = jnp.zeros((16,), _f32)
        return carry
    lax.fori_loop(0, 64 * 2, _zf, 0)

    for q in range(NQ):
        pltpu.sync_copy(xs_hbm.at[c, q].at[pl.ds(s * ROWS_PT, ROWS_PT)],
                        x_sh.at[pl.ds(s * ROWS_PT, ROWS_PT)])

        def _zcp(i, carry):
            pltpu.sync_copy(zbuf, agg_sh.at[pl.ds(s * ROWS_PT + i * 64, 64)])
            return carry
        lax.fori_loop(0, ROWS_PT // 64, _zcp, 0)
        plsc.subcore_barrier()

        for b in range(NBUF):
            pltpu.async_copy(x_sh.at[idx_s.at[b]], rows.at[b], gsem.at[b])

        @pl.loop(0, NCHUNK, step=NBUF)
        def _grp(j0):
            for b in range(NBUF):
                j = j0 + b
                pltpu.make_async_copy(
                    x_sh.at[idx_s.at[j]], rows.at[b], gsem.at[b]).wait()
                pltpu.async_copy(rows.at[b], agg_sh.at[idx_d.at[j]],
                                 ssem.at[b], add=True)
            for b in range(NBUF):
                j = j0 + b
                pltpu.make_async_copy(
                    rows.at[b], agg_sh.at[idx_d.at[j]], ssem.at[b]).wait()

                @pl.when(j + NBUF < NCHUNK)
                def _():
                    pltpu.async_copy(x_sh.at[idx_s.at[j + NBUF]],
                                     rows.at[b], gsem.at[b])

        plsc.subcore_barrier()
        pltpu.sync_copy(agg_sh.at[pl.ds(s * ROWS_PT, ROWS_PT)],
                        out_hbm.at[q].at[pl.ds(c * P + s * ROWS_PT,
                                               ROWS_PT)])
        plsc.subcore_barrier()


def _sc_agg(xs, src_all, dst_all):
    mesh = plsc.VectorSubcoreMesh(core_axis_name="c", subcore_axis_name="s",
                                  num_cores=NC, num_subcores=NS)
    return pl.kernel(
        _sc_agg_body,
        out_type=jax.ShapeDtypeStruct((NQ, 2 * P, Q), _f32),
        mesh=mesh,
        scratch_types=[
            pltpu.VMEM((NCHUNK, CW), jnp.int32),
            pltpu.VMEM((NCHUNK, CW), jnp.int32),
            pltpu.VMEM((NBUF, CW, Q), _f32),
            pltpu.VMEM((64, Q), _f32),
            pltpu.SemaphoreType.DMA((NBUF,)),
            pltpu.SemaphoreType.DMA((NBUF,)),
            pltpu.VMEM_SHARED((P, Q), _f32),
            pltpu.VMEM_SHARED((P, Q), _f32),
        ],
        compiler_params=pltpu.CompilerParams(use_tc_tiling_on_sc=False),
    )(xs, src_all, dst_all)


_BLK = 2560
_NBLK = 2 * P // _BLK
_row_spec = pl.BlockSpec((_BLK, D), lambda i: (i, 0))
_split_spec = pl.BlockSpec((1, NQ, _BLK, Q), lambda i: (i // NQ, 0, i % NQ, 0))
_agg_spec = pl.BlockSpec((NQ, _BLK, Q), lambda i: (0, i, 0))
_w_spec = pl.BlockSpec((D, D), lambda i: (0, 0))
_b_spec = pl.BlockSpec((1, D), lambda i: (0, 0))


def _split_out(o2_ref, y):
    for q in range(NQ):
        o2_ref[0, q] = y[:, q * Q:(q + 1) * Q]


def _pre_body(x_ref, w_ref, b_ref, o_ref, o2_ref):
    y = (jnp.dot(x_ref[...], w_ref[...], preferred_element_type=_f32)
         + b_ref[...])
    o_ref[...] = y
    _split_out(o2_ref, y)


def _pre(x, w, b):
    return pl.pallas_call(
        _pre_body,
        grid=(_NBLK,),
        in_specs=[_row_spec, _w_spec, _b_spec],
        out_specs=[_row_spec, _split_spec],
        out_shape=[jax.ShapeDtypeStruct((2 * P, D), _f32),
                   jax.ShapeDtypeStruct((2, NQ, P, Q), _f32)],
    )(x, w, b.reshape(1, D))


def _conv_mlp(x_ref, a_ref, r_ref, w1, b1, w2, b2):
    agg = jnp.concatenate([a_ref[q] for q in range(NQ)], axis=1)
    h = x_ref[...] + agg
    t = jnp.maximum(jnp.dot(h, w1[...], preferred_element_type=_f32)
                    + b1[...], 0.0)
    y = jnp.dot(t, w2[...], preferred_element_type=_f32) + b2[...]
    if r_ref is not None:
        y = y + r_ref[...]
    return jnp.maximum(y, 0.0)


def _conv_body(x_ref, a_ref, w1, b1, w2, b2, o_ref, o2_ref):
    y = _conv_mlp(x_ref, a_ref, None, w1, b1, w2, b2)
    o_ref[...] = y
    if o2_ref is not None:
        _split_out(o2_ref, y)


def _conv_body_res(x_ref, a_ref, r_ref, w1, b1, w2, b2, o_ref, o2_ref):
    y = _conv_mlp(x_ref, a_ref, r_ref, w1, b1, w2, b2)
    o_ref[...] = y
    if o2_ref is not None:
        _split_out(o2_ref, y)


def _conv(x, agg, w1, b1, w2, b2, res=None, split=True):
    if res is None:
        body, ins, args = _conv_body, [_row_spec, _agg_spec], (x, agg)
    else:
        body = _conv_body_res
        ins = [_row_spec, _agg_spec, _row_spec]
        args = (x, agg, res)
    out_specs = [_row_spec]
    out_shape = [jax.ShapeDtypeStruct((2 * P, D), _f32)]
    if split:
        out_specs.append(_split_spec)
        out_shape.append(jax.ShapeDtypeStruct((2, NQ, P, Q), _f32))
    else:
        body = (lambda *a, _b=body: _b(*a, None))
    return pl.pallas_call(
        body,
        grid=(_NBLK,),
        in_specs=ins + [_w_spec, _b_spec, _w_spec, _b_spec],
        out_specs=out_specs if split else out_specs[0],
        out_shape=out_shape if split else out_shape[0],
    )(*args, w1, b1.reshape(1, D), w2, b2.reshape(1, D))


_PBLK = 2560
_PNB = P // _PBLK


def _pool_body(b_ref, x0, x1, x2, x3, o_ref):
    rb = pl.program_id(1)

    @pl.when(rb == 0)
    def _():
        o_ref[...] = jnp.zeros_like(o_ref)

    ids = b_ref[0, 0, 0, :]
    oh = (ids[:, None] == lax.broadcasted_iota(jnp.int32, (_PBLK, N_GRAPHS), 1)
          ).astype(_f32)
    for k, xr in enumerate((x0, x1, x2, x3)):
        o_ref[0, :, k * D:(k + 1) * D] += jnp.dot(
            oh.T, xr[0], preferred_element_type=_f32)


def _pool(batch_r, embs):
    row_spec = pl.BlockSpec((1, _PBLK, D), lambda g, i: (g, i, 0))
    return pl.pallas_call(
        _pool_body,
        grid=(2, _PNB),
        in_specs=[pl.BlockSpec((1, 1, 1, _PBLK), lambda g, i: (g, i, 0, 0))]
        + [row_spec] * 4,
        out_specs=pl.BlockSpec((1, N_GRAPHS, 4 * D), lambda g, i: (g, 0, 0)),
        out_shape=jax.ShapeDtypeStruct((2, N_GRAPHS, 4 * D), _f32),
    )(batch_r, *embs)


def _head_body(p_ref, wp1, bp1, wp2, bp2, wm1, bm1, wm2, bm2, o_ref):
    def proj(pg):
        t = jnp.maximum(jnp.dot(pg, wp1[...], preferred_element_type=_f32)
                        + bp1[...], 0.0)
        return jnp.dot(t, wp2[...], preferred_element_type=_f32) + bp2[...]
    eg = proj(p_ref[0])
    eh = proj(p_ref[1])
    cat = jnp.concatenate((eg, eh), axis=1)
    t = jnp.maximum(jnp.dot(cat, wm1[...], preferred_element_type=_f32)
                    + bm1[...], 0.0)
    o_ref[...] = jnp.dot(t, wm2[...], preferred_element_type=_f32) + bm2[...]


def _head(pooled, wp1, bp1, wp2, bp2, wm1, bm1, wm2, bm2):
    def full(shape):
        return pl.BlockSpec(shape, lambda: tuple(0 for _ in shape))
    return pl.pallas_call(
        _head_body,
        in_specs=[full((2, N_GRAPHS, 4 * D)),
                  full((4 * D, D)), full((1, D)),
                  full((D, 64)), full((1, 64)),
                  full((2 * 64, 64)), full((1, 64)),
                  full((64, 1)), full((1, 1))],
        out_specs=full((N_GRAPHS, 1)),
        out_shape=jax.ShapeDtypeStruct((N_GRAPHS, 1), _f32),
    )(pooled, wp1, bp1.reshape(1, D), wp2, bp2.reshape(1, 64),
      wm1, bm1.reshape(1, 64), wm2, bm2.reshape(1, 1))


def _prep_edges(edge_index):
    npad = NS * EPT - edge_index.shape[1]
    sent = N_NODES + (jnp.arange(npad, dtype=jnp.int32) % (P - N_NODES))
    src = jnp.concatenate([edge_index[0], sent])
    dst = jnp.concatenate([edge_index[1], sent])
    return (src.reshape(NS, NCHUNK, CW), dst.reshape(NS, NCHUNK, CW))


def kernel(g_x, g_edge_index, g_batch, h_x, h_edge_index, h_batch,
           W_pre, b_pre, W1_0, b1_0, W2_0, b2_0, W1_1, b1_1, W2_1, b2_1,
           W1_2, b1_2, W2_2, b2_2, Wp1, bp1, Wp2, bp2, Wm1, bm1, Wm2, bm2):
    x_in = jnp.zeros((2 * P, D), _f32)
    x_in = x_in.at[:N_NODES].set(g_x)
    x_in = x_in.at[P:P + N_NODES].set(h_x)

    sg, dg = _prep_edges(g_edge_index)
    sh_, dh = _prep_edges(h_edge_index)
    src_all = jnp.stack([sg, sh_])
    dst_all = jnp.stack([dg, dh])

    x_pre, xs = _pre(x_in, W_pre, b_pre)
    convs = [(W1_0, b1_0, W2_0, b2_0),
             (W1_1, b1_1, W2_1, b2_1),
             (W1_2, b1_2, W2_2, b2_2)]
    x = x_pre
    embs = [x_pre]
    for i, (w1, b1, w2, b2) in enumerate(convs):
        agg = _sc_agg(xs, src_all, dst_all)
        r = _conv(x, agg, w1, b1, w2, b2,
                  res=x_pre if i == 1 else None, split=(i < 2))
        x, xs = r if i < 2 else (r, None)
        embs.append(x)

    pad_b = jnp.full((P - N_NODES,), N_GRAPHS, jnp.int32)
    batch_r = jnp.stack([jnp.concatenate([g_batch, pad_b]),
                         jnp.concatenate([h_batch, pad_b])]
                        ).reshape(2, _PNB, 1, _PBLK)
    pooled = _pool(batch_r, [e.reshape(2, P, D) for e in embs])
    out = _head(pooled, Wp1, bp1, Wp2, bp2, Wm1, bm1, Wm2, bm2)
    return out.reshape(-1)


# speedup vs baseline: 1.0699x; 1.0699x over previous
import jax
import jax.numpy as jnp
from jax import lax
from jax.experimental import pallas as pl
from jax.experimental.pallas import tpu as pltpu
from jax.experimental.pallas import tpu_sc as plsc

N_NODES = 10000
D = 128
NQ = 4
Q = D // NQ
N_GRAPHS = 64

NC = 2
NS = 16
P = 10240
EPT = 20480
NCHUNK = 160
CW = 128
ROWS_PT = P // NS

_f32 = jnp.float32


NBUF = 8


def _sc_agg_body(xs_hbm, src_hbm, dst_hbm, out_hbm,
                 idx_s, idx_d, rows, zbuf, gsem, ssem, x_sh, agg_sh):
    c = lax.axis_index("c")
    s = lax.axis_index("s")

    pltpu.sync_copy(src_hbm.at[c, s], idx_s)
    pltpu.sync_copy(dst_hbm.at[c, s], idx_d)

    def _zf(i, carry):
        zbuf[i // 2, pl.ds((i % 2) * 16, 16)] = jnp.zeros((16,), _f32)
        return carry
    lax.fori_loop(0, 64 * 2, _zf, 0)

    for q in range(NQ):
        pltpu.sync_copy(xs_hbm.at[c, q].at[pl.ds(s * ROWS_PT, ROWS_PT)],
                        x_sh.at[pl.ds(s * ROWS_PT, ROWS_PT)])

        def _zcp(i, carry):
            pltpu.sync_copy(zbuf, agg_sh.at[pl.ds(s * ROWS_PT + i * 64, 64)])
            return carry
        lax.fori_loop(0, ROWS_PT // 64, _zcp, 0)
        plsc.subcore_barrier()

        for b in range(NBUF):
            pltpu.async_copy(x_sh.at[idx_s.at[b]], rows.at[b], gsem.at[b])

        @pl.loop(0, NCHUNK, step=NBUF)
        def _grp(j0):
            for b in range(NBUF):
                j = j0 + b
                pltpu.make_async_copy(
                    x_sh.at[idx_s.at[j]], rows.at[b], gsem.at[b]).wait()
                pltpu.async_copy(rows.at[b], agg_sh.at[idx_d.at[j]],
                                 ssem.at[b], add=True)
            for b in range(NBUF):
                j = j0 + b
                pltpu.make_async_copy(
                    rows.at[b], agg_sh.at[idx_d.at[j]], ssem.at[b]).wait()

                @pl.when(j + NBUF < NCHUNK)
                def _():
                    pltpu.async_copy(x_sh.at[idx_s.at[j + NBUF]],
                                     rows.at[b], gsem.at[b])

        plsc.subcore_barrier()
        pltpu.sync_copy(agg_sh.at[pl.ds(s * ROWS_PT, ROWS_PT)],
                        out_hbm.at[q].at[pl.ds(c * P + s * ROWS_PT,
                                               ROWS_PT)])
        plsc.subcore_barrier()


def _sc_agg(xs, src_all, dst_all):
    mesh = plsc.VectorSubcoreMesh(core_axis_name="c", subcore_axis_name="s",
                                  num_cores=NC, num_subcores=NS)
    return pl.kernel(
        _sc_agg_body,
        out_type=jax.ShapeDtypeStruct((NQ, 2 * P, Q), _f32),
        mesh=mesh,
        scratch_types=[
            pltpu.VMEM((NCHUNK, CW), jnp.int32),
            pltpu.VMEM((NCHUNK, CW), jnp.int32),
            pltpu.VMEM((NBUF, CW, Q), _f32),
            pltpu.VMEM((64, Q), _f32),
            pltpu.SemaphoreType.DMA((NBUF,)),
            pltpu.SemaphoreType.DMA((NBUF,)),
            pltpu.VMEM_SHARED((P, Q), _f32),
            pltpu.VMEM_SHARED((P, Q), _f32),
        ],
        compiler_params=pltpu.CompilerParams(use_tc_tiling_on_sc=False),
    )(xs, src_all, dst_all)


_BLK = 2560
_NBLK = 2 * P // _BLK
_row_spec = pl.BlockSpec((_BLK, D), lambda i: (i, 0))
_split_spec = pl.BlockSpec((1, NQ, _BLK, Q), lambda i: (i // NQ, 0, i % NQ, 0))
_agg_spec = pl.BlockSpec((NQ, _BLK, Q), lambda i: (0, i, 0))
_w_spec = pl.BlockSpec((D, D), lambda i: (0, 0))
_b_spec = pl.BlockSpec((1, D), lambda i: (0, 0))


def _split_out(o2_ref, y):
    for q in range(NQ):
        o2_ref[0, q] = y[:, q * Q:(q + 1) * Q]


def _pre_body(x_ref, w_ref, b_ref, o_ref, o2_ref):
    y = (jnp.dot(x_ref[...], w_ref[...], preferred_element_type=_f32)
         + b_ref[...])
    o_ref[...] = y
    _split_out(o2_ref, y)


def _pre(x, w, b):
    return pl.pallas_call(
        _pre_body,
        grid=(_NBLK,),
        in_specs=[_row_spec, _w_spec, _b_spec],
        out_specs=[_row_spec, _split_spec],
        out_shape=[jax.ShapeDtypeStruct((2 * P, D), _f32),
                   jax.ShapeDtypeStruct((2, NQ, P, Q), _f32)],
    )(x, w, b.reshape(1, D))


def _conv_mlp(x_ref, a_ref, r_ref, w1, b1, w2, b2):
    agg = jnp.concatenate([a_ref[q] for q in range(NQ)], axis=1)
    h = x_ref[...] + agg
    t = jnp.maximum(jnp.dot(h, w1[...], preferred_element_type=_f32)
                    + b1[...], 0.0)
    y = jnp.dot(t, w2[...], preferred_element_type=_f32) + b2[...]
    if r_ref is not None:
        y = y + r_ref[...]
    return jnp.maximum(y, 0.0)


def _conv_body(x_ref, a_ref, w1, b1, w2, b2, o_ref, o2_ref):
    y = _conv_mlp(x_ref, a_ref, None, w1, b1, w2, b2)
    o_ref[...] = y
    if o2_ref is not None:
        _split_out(o2_ref, y)


def _conv_body_res(x_ref, a_ref, r_ref, w1, b1, w2, b2, o_ref, o2_ref):
    y = _conv_mlp(x_ref, a_ref, r_ref, w1, b1, w2, b2)
    o_ref[...] = y
    if o2_ref is not None:
        _split_out(o2_ref, y)


def _conv(x, agg, w1, b1, w2, b2, res=None, split=True):
    if res is None:
        body, ins, args = _conv_body, [_row_spec, _agg_spec], (x, agg)
    else:
        body = _conv_body_res
        ins = [_row_spec, _agg_spec, _row_spec]
        args = (x, agg, res)
    out_specs = [_row_spec]
    out_shape = [jax.ShapeDtypeStruct((2 * P, D), _f32)]
    if split:
        out_specs.append(_split_spec)
        out_shape.append(jax.ShapeDtypeStruct((2, NQ, P, Q), _f32))
    else:
        body = (lambda *a, _b=body: _b(*a, None))
    return pl.pallas_call(
        body,
        grid=(_NBLK,),
        in_specs=ins + [_w_spec, _b_spec, _w_spec, _b_spec],
        out_specs=out_specs if split else out_specs[0],
        out_shape=out_shape if split else out_shape[0],
    )(*args, w1, b1.reshape(1, D), w2, b2.reshape(1, D))


_PBLK = 2560
_PNB = P // _PBLK


def _pool_body(b_ref, x0, x1, x2, x3, o_ref):
    rb = pl.program_id(1)

    @pl.when(rb == 0)
    def _():
        o_ref[...] = jnp.zeros_like(o_ref)

    ids = b_ref[0, 0, 0, :]
    oh = (ids[:, None] == lax.broadcasted_iota(jnp.int32, (_PBLK, N_GRAPHS), 1)
          ).astype(_f32)
    for k, xr in enumerate((x0, x1, x2, x3)):
        o_ref[0, :, k * D:(k + 1) * D] += jnp.dot(
            oh.T, xr[0], preferred_element_type=_f32)


def _pool(batch_r, embs):
    row_spec = pl.BlockSpec((1, _PBLK, D), lambda g, i: (g, i, 0))
    return pl.pallas_call(
        _pool_body,
        grid=(2, _PNB),
        in_specs=[pl.BlockSpec((1, 1, 1, _PBLK), lambda g, i: (g, i, 0, 0))]
        + [row_spec] * 4,
        out_specs=pl.BlockSpec((1, N_GRAPHS, 4 * D), lambda g, i: (g, 0, 0)),
        out_shape=jax.ShapeDtypeStruct((2, N_GRAPHS, 4 * D), _f32),
    )(batch_r, *embs)


def _head_body(p_ref, wp1, bp1, wp2, bp2, wm1, bm1, wm2, bm2, o_ref):
    def proj(pg):
        t = jnp.maximum(jnp.dot(pg, wp1[...], preferred_element_type=_f32)
                        + bp1[...], 0.0)
        return jnp.dot(t, wp2[...], preferred_element_type=_f32) + bp2[...]
    eg = proj(p_ref[0])
    eh = proj(p_ref[1])
    cat = jnp.concatenate((eg, eh), axis=1)
    t = jnp.maximum(jnp.dot(cat, wm1[...], preferred_element_type=_f32)
                    + bm1[...], 0.0)
    o_ref[...] = jnp.dot(t, wm2[...], preferred_element_type=_f32) + bm2[...]


def _head(pooled, wp1, bp1, wp2, bp2, wm1, bm1, wm2, bm2):
    def full(shape):
        return pl.BlockSpec(shape, lambda: tuple(0 for _ in shape))
    return pl.pallas_call(
        _head_body,
        in_specs=[full((2, N_GRAPHS, 4 * D)),
                  full((4 * D, D)), full((1, D)),
                  full((D, 64)), full((1, 64)),
                  full((2 * 64, 64)), full((1, 64)),
                  full((64, 1)), full((1, 1))],
        out_specs=full((N_GRAPHS, 1)),
        out_shape=jax.ShapeDtypeStruct((N_GRAPHS, 1), _f32),
    )(pooled, wp1, bp1.reshape(1, D), wp2, bp2.reshape(1, 64),
      wm1, bm1.reshape(1, 64), wm2, bm2.reshape(1, 1))


def _prep_edges(edge_index):
    npad = NS * EPT - edge_index.shape[1]
    sent = N_NODES + (jnp.arange(npad, dtype=jnp.int32) % (P - N_NODES))
    src = jnp.concatenate([edge_index[0], sent])
    dst = jnp.concatenate([edge_index[1], sent])
    return (src.reshape(NS, NCHUNK, CW), dst.reshape(NS, NCHUNK, CW))


def kernel(g_x, g_edge_index, g_batch, h_x, h_edge_index, h_batch,
           W_pre, b_pre, W1_0, b1_0, W2_0, b2_0, W1_1, b1_1, W2_1, b2_1,
           W1_2, b1_2, W2_2, b2_2, Wp1, bp1, Wp2, bp2, Wm1, bm1, Wm2, bm2):
    x_in = jnp.zeros((2 * P, D), _f32)
    x_in = x_in.at[:N_NODES].set(g_x)
    x_in = x_in.at[P:P + N_NODES].set(h_x)

    sg, dg = _prep_edges(g_edge_index)
    sh_, dh = _prep_edges(h_edge_index)
    src_all = jnp.stack([sg, sh_])
    dst_all = jnp.stack([dg, dh])

    x_pre, xs = _pre(x_in, W_pre, b_pre)
    convs = [(W1_0, b1_0, W2_0, b2_0),
             (W1_1, b1_1, W2_1, b2_1),
             (W1_2, b1_2, W2_2, b2_2)]
    x = x_pre
    embs = [x_pre]
    for i, (w1, b1, w2, b2) in enumerate(convs):
        agg = _sc_agg(xs, src_all, dst_all)
        r = _conv(x, agg, w1, b1, w2, b2,
                  res=x_pre if i == 1 else None, split=(i < 2))
        x, xs = r if i < 2 else (r, None)
        embs.append(x)

    pad_b = jnp.full((P - N_NODES,), N_GRAPHS, jnp.int32)
    batch_r = jnp.stack([jnp.concatenate([g_batch, pad_b]),
                         jnp.concatenate([h_batch, pad_b])]
                        ).reshape(2, _PNB, 1, _PBLK)
    pooled = _pool(batch_r, [e.reshape(2, P, D) for e in embs])
    out = _head(pooled, Wp1, bp1, Wp2, bp2, Wm1, bm1, Wm2, bm2)
    return out.reshape(-1)
